# baseline (speedup 1.0000x reference)
import functools

import jax
import jax.numpy as jnp
from jax import lax
from jax.experimental import pallas as pl
from jax.experimental.pallas import tpu as pltpu
from jax.experimental.pallas import tpu_sc as plsc

B = 16384
N_WORDS = 3
WORD_DIM = 32
HIDDEN = 256
OUT = 64
WIDE = 128
PACK = WIDE // WORD_DIM
NSLOT = 2 * PACK
S = 1 << 17
VOCAB = 1000000

WB = 8192
TGRID = S // WB
CPB = S // WB

_LAST_BLK = (VOCAB - 1) // WB


def _bf16_bits(x_i32):
    lsb = jnp.bitwise_and(lax.shift_right_logical(x_i32, 16), 1)
    return lax.shift_right_logical(x_i32 + 0x7FFF + lsb, 16)


def _tr_body(x0, x1, x2, x3, x4, x5, x6, x7, out):
    lo = jnp.concatenate([x0[...], x1[...], x2[...], x3[...]], axis=0).T
    hi = jnp.concatenate([x4[...], x5[...], x6[...], x7[...]], axis=0).T
    lob = _bf16_bits(lax.bitcast_convert_type(lo, jnp.int32))
    hib = _bf16_bits(lax.bitcast_convert_type(hi, jnp.int32))
    out[...] = jnp.bitwise_or(lax.shift_left(hib, 16), lob)


def _tr_spec(j):
    return pl.BlockSpec(
        (WORD_DIM, WB),
        lambda i, j=j: (0, jnp.minimum(i + j * CPB, _LAST_BLK)))


_transpose = pl.pallas_call(
    _tr_body,
    grid=(TGRID,),
    in_specs=[_tr_spec(j) for j in range(NSLOT)],
    out_specs=pl.BlockSpec((WB, WIDE), lambda i: (i, 0)),
    out_shape=jax.ShapeDtypeStruct((S, WIDE), jnp.int32),
)

NC = 2
NS = 16
NW = NC * NS
CHUNK = 128
NBUF = 6

_sc_mesh = plsc.VectorSubcoreMesh(core_axis_name="c", subcore_axis_name="s")


def _make_gather(n_feat):
    rows = n_feat * (B // NW)
    nch = rows // CHUNK

    @functools.partial(
        pl.kernel,
        mesh=_sc_mesh,
        out_type=[jax.ShapeDtypeStruct((B, WIDE), jnp.int32)
                  for _ in range(n_feat)],
        scratch_types=[
            pltpu.VMEM((rows,), jnp.int32),
            pltpu.VMEM((NBUF, CHUNK, WIDE), jnp.int32),
        ] + [pltpu.SemaphoreType.DMA for _ in range(2 * NBUF)],
        compiler_params=pltpu.CompilerParams(use_tc_tiling_on_sc=True),
    )
    def gather(wide_idx, table, *rest):
        outs = rest[:n_feat]
        idx_v, bufs = rest[n_feat:n_feat + 2]
        sems = rest[n_feat + 2:]
        gsem = sems[:NBUF]
        osem = sems[NBUF:]
        wid = lax.axis_index("s") * NC + lax.axis_index("c")
        rpw = B // NW
        ncpf = rpw // CHUNK
        for f in range(n_feat):
            pltpu.sync_copy(wide_idx.at[pl.ds(f * B + wid * rpw, rpw)],
                            idx_v.at[pl.ds(f * rpw, rpw)])

        def fire_gather(j):
            return pltpu.async_copy(
                table.at[idx_v.at[pl.ds(j * CHUNK, CHUNK)]],
                bufs.at[j % NBUF], gsem[j % NBUF])

        def fire_out(j):
            f, c = divmod(j, ncpf)
            return pltpu.async_copy(
                bufs.at[j % NBUF],
                outs[f].at[pl.ds(wid * rpw + c * CHUNK, CHUNK)],
                osem[j % NBUF])

        g = [None] * nch
        o = [None] * nch
        for j in range(nch):
            if j >= NBUF:
                o[j - NBUF].wait()
            g[j] = fire_gather(j)
            k = j - (NBUF - 1)
            if k >= 0:
                g[k].wait()
                o[k] = fire_out(k)
        for k in range(max(0, nch - NBUF + 1), nch):
            g[k].wait()
            o[k] = fire_out(k)
        for k in range(max(0, nch - NBUF), nch):
            o[k].wait()

    return gather


_gather_words = _make_gather(N_WORDS)
_gather_tags = _make_gather(1)

BLK = 4096
NFEAT = N_WORDS + 1


def _mlp_body(w0_ref, w1_ref, w2_ref, t_ref, off_ref,
              ws_ref, bh_ref, wo_ref, bo_ref, out_ref):
    offs = off_ref[...]
    lane_grp = (jax.lax.broadcasted_iota(jnp.int32, (1, WIDE), 1)
                // WORD_DIM).astype(jnp.float32)
    h = bh_ref[...]
    wides = (w0_ref, w1_ref, w2_ref, t_ref)
    for k in range(NFEAT):
        off = offs[:, k:k + 1]
        use_hi = off >= float(PACK) - 0.5
        grp = off - jnp.where(use_hi, float(PACK), 0.0)
        x = wides[k][...]
        lo_f = lax.bitcast_convert_type(lax.shift_left(x, 16), jnp.float32)
        hi_f = lax.bitcast_convert_type(
            jnp.bitwise_and(x, jnp.int32(-65536)), jnp.float32)
        val = jnp.where(use_hi, hi_f, lo_f)
        mask = (grp == lane_grp).astype(jnp.float32)
        h = h + jnp.dot(val * mask,
                        ws_ref[WIDE * k:WIDE * (k + 1), :],
                        preferred_element_type=jnp.float32)
    h = jnp.maximum(h, 0.0)
    out_ref[...] = (jnp.dot(h, wo_ref[...],
                            preferred_element_type=jnp.float32)
                    + bo_ref[...]).T


_mlp = pl.pallas_call(
    _mlp_body,
    grid=(B // BLK,),
    in_specs=[
        pl.BlockSpec((BLK, WIDE), lambda i: (i, 0)),
        pl.BlockSpec((BLK, WIDE), lambda i: (i, 0)),
        pl.BlockSpec((BLK, WIDE), lambda i: (i, 0)),
        pl.BlockSpec((BLK, WIDE), lambda i: (i, 0)),
        pl.BlockSpec((BLK, NFEAT), lambda i: (i, 0)),
        pl.BlockSpec((NFEAT * WIDE, HIDDEN), lambda i: (0, 0)),
        pl.BlockSpec((1, HIDDEN), lambda i: (0, 0)),
        pl.BlockSpec((HIDDEN, OUT), lambda i: (0, 0)),
        pl.BlockSpec((1, OUT), lambda i: (0, 0)),
    ],
    out_specs=pl.BlockSpec((OUT, BLK), lambda i: (0, i)),
    out_shape=jax.ShapeDtypeStruct((OUT, B), jnp.float32),
)


def kernel(features, word_table, tag_table, W_h, b_h, W_o, b_o):
    widx = features & (S - 1)
    word_idx = widx[:, :N_WORDS].T.reshape(-1)
    tag_idx = widx[:, N_WORDS]
    offs = (features >> 17).astype(jnp.float32)
    w_stack = jnp.concatenate(
        [jnp.tile(W_h[WORD_DIM * k:WORD_DIM * (k + 1), :], (PACK, 1))
         for k in range(NFEAT)], axis=0)
    word_tw = _transpose(*([word_table.T] * NSLOT))
    w0, w1, w2 = _gather_words(word_idx, word_tw)
    tag_tw = _transpose(*([tag_table.T] * NSLOT))
    t, = _gather_tags(tag_idx, tag_tw)
    out_t = _mlp(w0, w1, w2, t, offs, w_stack, b_h.reshape(1, HIDDEN),
                 W_o, b_o.reshape(1, OUT))
    return out_t.T


# speedup vs baseline: 1.0172x; 1.0172x over previous
import functools

import jax
import jax.numpy as jnp
from jax import lax
from jax.experimental import pallas as pl
from jax.experimental.pallas import tpu as pltpu
from jax.experimental.pallas import tpu_sc as plsc

B = 16384
N_WORDS = 3
WORD_DIM = 32
HIDDEN = 256
OUT = 64
WIDE = 128
PACK = WIDE // WORD_DIM
NSLOT = 2 * PACK
S = 1 << 17
VOCAB = 1000000

WB = 8192
TGRID = S // WB
CPB = S // WB

_LAST_BLK = (VOCAB - 1) // WB


def _bf16_bits(x_i32):
    lsb = jnp.bitwise_and(lax.shift_right_logical(x_i32, 16), 1)
    return lax.shift_right_logical(x_i32 + 0x7FFF + lsb, 16)


def _tr_body(x0, x1, x2, x3, x4, x5, x6, x7, out):
    lo = jnp.concatenate([x0[...], x1[...], x2[...], x3[...]], axis=0).T
    hi = jnp.concatenate([x4[...], x5[...], x6[...], x7[...]], axis=0).T
    lob = _bf16_bits(lax.bitcast_convert_type(lo, jnp.int32))
    hib = _bf16_bits(lax.bitcast_convert_type(hi, jnp.int32))
    out[...] = jnp.bitwise_or(lax.shift_left(hib, 16), lob)


def _tr_spec(j):
    return pl.BlockSpec(
        (WORD_DIM, WB),
        lambda i, j=j: (0, jnp.minimum(i + j * CPB, _LAST_BLK)))


_transpose = pl.pallas_call(
    _tr_body,
    grid=(TGRID,),
    in_specs=[_tr_spec(j) for j in range(NSLOT)],
    out_specs=pl.BlockSpec((WB, WIDE), lambda i: (i, 0)),
    out_shape=jax.ShapeDtypeStruct((S, WIDE), jnp.int32),
)

NC = 2
NS = 16
NW = NC * NS
CHUNK = 128
NBUF = 6

_sc_mesh = plsc.VectorSubcoreMesh(core_axis_name="c", subcore_axis_name="s")


def _make_gather(n_feat):
    rows = n_feat * (B // NW)
    nch = rows // CHUNK

    @functools.partial(
        pl.kernel,
        mesh=_sc_mesh,
        out_type=[jax.ShapeDtypeStruct((B, WIDE), jnp.int32)
                  for _ in range(n_feat)],
        scratch_types=[
            pltpu.VMEM((rows,), jnp.int32),
            pltpu.VMEM((NBUF, CHUNK, WIDE), jnp.int32),
        ] + [pltpu.SemaphoreType.DMA for _ in range(2 * NBUF)],
        compiler_params=pltpu.CompilerParams(use_tc_tiling_on_sc=True),
    )
    def gather(wide_idx, table, *rest):
        outs = rest[:n_feat]
        idx_v, bufs = rest[n_feat:n_feat + 2]
        sems = rest[n_feat + 2:]
        gsem = sems[:NBUF]
        osem = sems[NBUF:]
        wid = lax.axis_index("s") * NC + lax.axis_index("c")
        rpw = B // NW
        ncpf = rpw // CHUNK
        for f in range(n_feat):
            pltpu.sync_copy(wide_idx.at[pl.ds(f * B + wid * rpw, rpw)],
                            idx_v.at[pl.ds(f * rpw, rpw)])

        def fire_gather(j):
            return pltpu.async_copy(
                table.at[idx_v.at[pl.ds(j * CHUNK, CHUNK)]],
                bufs.at[j % NBUF], gsem[j % NBUF])

        def fire_out(j):
            f, c = divmod(j, ncpf)
            return pltpu.async_copy(
                bufs.at[j % NBUF],
                outs[f].at[pl.ds(wid * rpw + c * CHUNK, CHUNK)],
                osem[j % NBUF])

        g = [None] * nch
        o = [None] * nch
        for j in range(nch):
            if j >= NBUF:
                o[j - NBUF].wait()
            g[j] = fire_gather(j)
            k = j - (NBUF - 1)
            if k >= 0:
                g[k].wait()
                o[k] = fire_out(k)
        for k in range(max(0, nch - NBUF + 1), nch):
            g[k].wait()
            o[k] = fire_out(k)
        for k in range(max(0, nch - NBUF), nch):
            o[k].wait()

    return gather


_gather_words = _make_gather(N_WORDS)
_gather_tags = _make_gather(1)

BLK = 4096
NFEAT = N_WORDS + 1


def _mlp_body(w0_ref, w1_ref, w2_ref, t_ref, feat_ref,
              wh_ref, bh_ref, wo_ref, bo_ref, out_ref):
    feat = feat_ref[...]
    lane_grp = jax.lax.broadcasted_iota(jnp.int32, (1, WIDE), 1) // WORD_DIM
    h = bh_ref[...]
    wides = (w0_ref, w1_ref, w2_ref, t_ref)
    for k in range(NFEAT):
        off = lax.shift_right_logical(feat[:, k:k + 1], 17)
        use_hi = off >= PACK
        grp = jnp.bitwise_and(off, PACK - 1)
        x = wides[k][...]
        lo_f = lax.bitcast_convert_type(lax.shift_left(x, 16), jnp.float32)
        hi_f = lax.bitcast_convert_type(
            jnp.bitwise_and(x, jnp.int32(-65536)), jnp.float32)
        val = jnp.where(use_hi, hi_f, lo_f)
        mask = (grp == lane_grp).astype(jnp.float32)
        wstk = jnp.concatenate(
            [wh_ref[WORD_DIM * k:WORD_DIM * (k + 1), :]] * PACK, axis=0)
        h = h + jnp.dot(val * mask, wstk,
                        preferred_element_type=jnp.float32)
    h = jnp.maximum(h, 0.0)
    out_ref[...] = (jnp.dot(h, wo_ref[...],
                            preferred_element_type=jnp.float32)
                    + bo_ref[...]).T


_mlp = pl.pallas_call(
    _mlp_body,
    grid=(B // BLK,),
    in_specs=[
        pl.BlockSpec((BLK, WIDE), lambda i: (i, 0)),
        pl.BlockSpec((BLK, WIDE), lambda i: (i, 0)),
        pl.BlockSpec((BLK, WIDE), lambda i: (i, 0)),
        pl.BlockSpec((BLK, WIDE), lambda i: (i, 0)),
        pl.BlockSpec((BLK, NFEAT), lambda i: (i, 0)),
        pl.BlockSpec((NFEAT * WORD_DIM, HIDDEN), lambda i: (0, 0)),
        pl.BlockSpec((1, HIDDEN), lambda i: (0, 0)),
        pl.BlockSpec((HIDDEN, OUT), lambda i: (0, 0)),
        pl.BlockSpec((1, OUT), lambda i: (0, 0)),
    ],
    out_specs=pl.BlockSpec((OUT, BLK), lambda i: (0, i)),
    out_shape=jax.ShapeDtypeStruct((OUT, B), jnp.float32),
)


def kernel(features, word_table, tag_table, W_h, b_h, W_o, b_o):
    widx = features & (S - 1)
    word_idx = widx[:, :N_WORDS].T.reshape(-1)
    tag_idx = widx[:, N_WORDS]
    word_tw = _transpose(*([word_table.T] * NSLOT))
    w0, w1, w2 = _gather_words(word_idx, word_tw)
    tag_tw = _transpose(*([tag_table.T] * NSLOT))
    t, = _gather_tags(tag_idx, tag_tw)
    out_t = _mlp(w0, w1, w2, t, features, W_h, b_h.reshape(1, HIDDEN),
                 W_o, b_o.reshape(1, OUT))
    return out_t.T
